# baseline (speedup 1.0000x reference)
import functools

import jax
import jax.numpy as jnp
import numpy as np
from jax.experimental import pallas as pl
from jax.experimental.pallas import tpu as pltpu

_EPS = 0.01
_N = 1_000_000
_ROWS = 32
_SUB = 8
_LANE = _N // _SUB
_TINY = np.float32(np.finfo(np.float32).tiny)

_KS0 = np.uint32(0)
_KS1 = np.uint32(42)
_KS2 = np.uint32(int(_KS0) ^ int(_KS1) ^ 0x1BD11BDA)
_KS = (_KS0, _KS1, _KS2)
_ROT_A = (13, 15, 26, 6)
_ROT_B = (17, 29, 16, 24)


def _threefry_bits(x1):
    x0 = jnp.zeros_like(x1) + _KS0
    x1 = x1 + _KS1
    for s, rots in enumerate((_ROT_A, _ROT_B, _ROT_A, _ROT_B, _ROT_A), start=1):
        for r in rots:
            x0 = x0 + x1
            x1 = ((x1 << np.uint32(r)) | (x1 >> np.uint32(32 - r))) ^ x0
        x0 = x0 + _KS[s % 3]
        x1 = x1 + _KS[(s + 1) % 3] + np.uint32(s)
    return x0 ^ x1


def _row_kernel(x_ref, o_ref):
    r = pl.program_id(0)
    x = x_ref[0]
    m = jnp.max(x)
    e = jnp.exp(x - m)
    s = jnp.sum(e)
    c = s * np.float32(_EPS / (1.0 - _EPS) / _N)

    flat = (jax.lax.broadcasted_iota(jnp.uint32, (_SUB, _LANE), 0) * np.uint32(_LANE)
            + jax.lax.broadcasted_iota(jnp.uint32, (_SUB, _LANE), 1))
    cnt = flat + r.astype(jnp.uint32) * np.uint32(_N)
    bits = _threefry_bits(cnt)

    f = jax.lax.bitcast_convert_type(
        np.uint32(0x3F800000) | (bits >> np.uint32(9)), jnp.float32) - np.float32(1.0)
    u = jnp.where(f > np.float32(0.0), f, _TINY)
    t = -jnp.log(u)

    ratio = (e + c) / t
    gmax = jnp.max(ratio)
    flat_i = flat.astype(jnp.int32)
    cand = jnp.where(ratio == gmax, flat_i, np.int32(2**31 - 1))
    am = jnp.min(cand)
    o_ref[0] = jnp.where(flat_i == am, np.float32(1.0), np.float32(0.0))


@functools.partial(jax.jit, static_argnames=("interpret",))
def kernel(logits, interpret=False):
    x3 = logits.reshape(_ROWS, _SUB, _LANE)
    out = pl.pallas_call(
        _row_kernel,
        grid=(_ROWS,),
        in_specs=[pl.BlockSpec((1, _SUB, _LANE), lambda r: (r, 0, 0))],
        out_specs=pl.BlockSpec((1, _SUB, _LANE), lambda r: (r, 0, 0)),
        out_shape=jax.ShapeDtypeStruct((_ROWS, _SUB, _LANE), jnp.float32),
        compiler_params=pltpu.CompilerParams(
            dimension_semantics=("arbitrary",),
            vmem_limit_bytes=100 * 1024 * 1024,
        ),
        interpret=interpret,
    )(x3)
    return out.reshape(_ROWS, _N)


# speedup vs baseline: 1.2468x; 1.2468x over previous
import functools

import jax
import jax.numpy as jnp
import numpy as np
from jax.experimental import pallas as pl
from jax.experimental.pallas import tpu as pltpu

_EPS = 0.01
_N = 1_000_000
_ROWS = 32
_CHUNKS = 125
_SUB = 8
_LANE = 1000
_CHUNK_ELEMS = _SUB * _LANE
_TINY = np.float32(np.finfo(np.float32).tiny)
_LOG2E = np.float32(1.4426950408889634)
_NEG_INV_LN2 = np.float32(-1.4426950408889634)

_KS1 = np.uint32(42)
_KS2 = np.uint32(42 ^ 0x1BD11BDA)
_INJECT = (
    (np.uint32(42), np.uint32(int(_KS2) + 1)),
    (_KS2, np.uint32(0 + 2)),
    (None, np.uint32(42 + 3)),
    (np.uint32(42), np.uint32(int(_KS2) + 4)),
    (_KS2, np.uint32(0 + 5)),
)
_ROT_A = (13, 15, 26, 6)
_ROT_B = (17, 29, 16, 24)
_GROUPS = (_ROT_A, _ROT_B, _ROT_A, _ROT_B, _ROT_A)


def _rotl(v, r):
    return (v << np.uint32(r)) | (v >> np.uint32(32 - r))


def _threefry_bits(cnt):
    x1 = cnt
    x0 = x1
    x1 = _rotl(x1, _ROT_A[0]) ^ x1
    for r in _ROT_A[1:]:
        x0 = x0 + x1
        x1 = _rotl(x1, r) ^ x0
    for g in range(1, 5):
        a, b = _INJECT[g - 1]
        if a is not None:
            x0 = x0 + a
        x1 = x1 + b
        for r in _GROUPS[g]:
            x0 = x0 + x1
            x1 = _rotl(x1, r) ^ x0
    a, b = _INJECT[4]
    x0 = x0 + a
    x1 = x1 + b
    return x0 ^ x1


def _row_kernel(x_ref, o_ref):
    r = pl.program_id(0)

    def max_body(c, mx):
        return jnp.maximum(mx, x_ref[0, c])

    mx = jax.lax.fori_loop(
        0, _CHUNKS, max_body,
        jnp.full((_SUB, _LANE), -jnp.inf, jnp.float32))
    m = jnp.max(mx)
    m2 = m * _LOG2E

    def sum_body(c, acc):
        return acc + jnp.exp2(x_ref[0, c] * _LOG2E - m2)

    acc = jax.lax.fori_loop(
        0, _CHUNKS, sum_body, jnp.zeros((_SUB, _LANE), jnp.float32))
    s = jnp.sum(acc)
    c_mix = s * np.float32(_EPS / (1.0 - _EPS) / _N)

    base2d = (jax.lax.broadcasted_iota(jnp.int32, (_SUB, _LANE), 0) * _LANE
              + jax.lax.broadcasted_iota(jnp.int32, (_SUB, _LANE), 1))
    cnt0 = base2d.astype(jnp.uint32) + (
        r.astype(jnp.uint32) * np.uint32(_N) + np.uint32(42))

    def score_body(c, carry):
        best, bflat = carry
        x = x_ref[0, c]
        off = jnp.uint32(c) * np.uint32(_CHUNK_ELEMS)
        bits = _threefry_bits(cnt0 + off)
        f = jax.lax.bitcast_convert_type(
            np.uint32(0x3F800000) | (bits >> np.uint32(9)), jnp.float32)
        u = jnp.maximum(f - np.float32(1.0), _TINY)
        l2 = jnp.log2(u)
        w = jnp.exp2(x * _LOG2E - m2) + c_mix
        ratio = (w * _NEG_INV_LN2) / l2
        upd = ratio > best
        best = jnp.where(upd, ratio, best)
        flat = base2d + c * _CHUNK_ELEMS
        bflat = jnp.where(upd, flat, bflat)
        o_ref[0, c] = jnp.zeros((_SUB, _LANE), jnp.float32)
        return best, bflat

    best, bflat = jax.lax.fori_loop(
        0, _CHUNKS, score_body,
        (jnp.zeros((_SUB, _LANE), jnp.float32),
         jnp.zeros((_SUB, _LANE), jnp.int32)))

    bestv = jnp.max(best)
    cand = jnp.where(best == bestv, bflat, np.int32(2**31 - 1))
    am = jnp.min(cand)

    chunk = am // _CHUNK_ELEMS
    rem = am - chunk * _CHUNK_ELEMS
    sub = rem // _LANE
    lane = rem - sub * _LANE
    li = jax.lax.broadcasted_iota(jnp.int32, (1, _LANE), 1)
    o_ref[0, chunk, pl.ds(sub, 1), :] = jnp.where(
        li == lane, np.float32(1.0), np.float32(0.0))


@functools.partial(jax.jit, static_argnames=("interpret",))
def kernel(logits, interpret=False):
    x4 = logits.reshape(_ROWS, _CHUNKS, _SUB, _LANE)
    out = pl.pallas_call(
        _row_kernel,
        grid=(_ROWS,),
        in_specs=[pl.BlockSpec((1, _CHUNKS, _SUB, _LANE), lambda r: (r, 0, 0, 0))],
        out_specs=pl.BlockSpec((1, _CHUNKS, _SUB, _LANE), lambda r: (r, 0, 0, 0)),
        out_shape=jax.ShapeDtypeStruct((_ROWS, _CHUNKS, _SUB, _LANE), jnp.float32),
        compiler_params=pltpu.CompilerParams(
            dimension_semantics=("arbitrary",),
            vmem_limit_bytes=100 * 1024 * 1024,
        ),
        interpret=interpret,
    )(x4)
    return out.reshape(_ROWS, _N)


# speedup vs baseline: 1.2694x; 1.0182x over previous
import functools

import jax
import jax.numpy as jnp
import numpy as np
from jax.experimental import pallas as pl
from jax.experimental.pallas import tpu as pltpu

_EPS = 0.01
_N = 1_000_000
_ROWS = 32
_CHUNKS = 50
_SUB = 8
_LANE = 2500
_CHUNK_ELEMS = _SUB * _LANE
_TINY = np.float32(np.finfo(np.float32).tiny)
_LOG2E = np.float32(1.4426950408889634)
_NEG_INV_LN2 = np.float32(-1.4426950408889634)

_KS1 = np.uint32(42)
_KS2 = np.uint32(42 ^ 0x1BD11BDA)
_INJECT = (
    (np.uint32(42), np.uint32(int(_KS2) + 1)),
    (_KS2, np.uint32(0 + 2)),
    (None, np.uint32(42 + 3)),
    (np.uint32(42), np.uint32(int(_KS2) + 4)),
    (_KS2, np.uint32(0 + 5)),
)
_ROT_A = (13, 15, 26, 6)
_ROT_B = (17, 29, 16, 24)
_GROUPS = (_ROT_A, _ROT_B, _ROT_A, _ROT_B, _ROT_A)


def _rotl(v, r):
    return (v << np.uint32(r)) | (v >> np.uint32(32 - r))


def _threefry_bits(cnt):
    x1 = cnt
    x0 = x1
    x1 = _rotl(x1, _ROT_A[0]) ^ x1
    for r in _ROT_A[1:]:
        x0 = x0 + x1
        x1 = _rotl(x1, r) ^ x0
    for g in range(1, 5):
        a, b = _INJECT[g - 1]
        if a is not None:
            x0 = x0 + a
        x1 = x1 + b
        for r in _GROUPS[g]:
            x0 = x0 + x1
            x1 = _rotl(x1, r) ^ x0
    a, b = _INJECT[4]
    x0 = x0 + a
    x1 = x1 + b
    return x0 ^ x1


def _row_kernel(x_ref, o_ref):
    r = pl.program_id(0)

    def max_body(c, mx):
        return jnp.maximum(mx, x_ref[0, c])

    mx = jax.lax.fori_loop(
        0, _CHUNKS, max_body,
        jnp.full((_SUB, _LANE), -jnp.inf, jnp.float32))
    m = jnp.max(mx)
    m2 = m * _LOG2E

    def sum_body(c, acc):
        return acc + jnp.exp2(x_ref[0, c] * _LOG2E - m2)

    acc = jax.lax.fori_loop(
        0, _CHUNKS, sum_body, jnp.zeros((_SUB, _LANE), jnp.float32))
    s = jnp.sum(acc)
    c_mix = s * np.float32(_EPS / (1.0 - _EPS) / _N)

    base2d = (jax.lax.broadcasted_iota(jnp.int32, (_SUB, _LANE), 0) * _LANE
              + jax.lax.broadcasted_iota(jnp.int32, (_SUB, _LANE), 1))
    cnt0 = base2d.astype(jnp.uint32) + (
        r.astype(jnp.uint32) * np.uint32(_N) + np.uint32(42))

    def score_body(c, carry):
        best, bchunk = carry
        x = x_ref[0, c]
        off = jnp.uint32(c) * np.uint32(_CHUNK_ELEMS)
        bits = _threefry_bits(cnt0 + off)
        f = jax.lax.bitcast_convert_type(
            np.uint32(0x3F800000) | (bits >> np.uint32(9)), jnp.float32)
        u = jnp.maximum(f - np.float32(1.0), _TINY)
        l2 = jnp.log2(u)
        w = jnp.exp2(x * _LOG2E - m2) + c_mix
        ratio = (w * _NEG_INV_LN2) / l2
        upd = ratio > best
        best = jnp.where(upd, ratio, best)
        bchunk = jnp.where(upd, c, bchunk)
        o_ref[0, c] = jnp.zeros((_SUB, _LANE), jnp.float32)
        return best, bchunk

    best, bchunk = jax.lax.fori_loop(
        0, _CHUNKS, score_body,
        (jnp.zeros((_SUB, _LANE), jnp.float32),
         jnp.zeros((_SUB, _LANE), jnp.int32)))

    bestv = jnp.max(best)
    cand = jnp.where(best == bestv,
                     bchunk * _CHUNK_ELEMS + base2d, np.int32(2**31 - 1))
    am = jnp.min(cand)

    chunk = am // _CHUNK_ELEMS
    rem = am - chunk * _CHUNK_ELEMS
    sub = rem // _LANE
    lane = rem - sub * _LANE
    li = jax.lax.broadcasted_iota(jnp.int32, (1, _LANE), 1)
    o_ref[0, chunk, pl.ds(sub, 1), :] = jnp.where(
        li == lane, np.float32(1.0), np.float32(0.0))


@functools.partial(jax.jit, static_argnames=("interpret",))
def kernel(logits, interpret=False):
    x4 = logits.reshape(_ROWS, _CHUNKS, _SUB, _LANE)
    out = pl.pallas_call(
        _row_kernel,
        grid=(_ROWS,),
        in_specs=[pl.BlockSpec((1, _CHUNKS, _SUB, _LANE), lambda r: (r, 0, 0, 0))],
        out_specs=pl.BlockSpec((1, _CHUNKS, _SUB, _LANE), lambda r: (r, 0, 0, 0)),
        out_shape=jax.ShapeDtypeStruct((_ROWS, _CHUNKS, _SUB, _LANE), jnp.float32),
        compiler_params=pltpu.CompilerParams(
            dimension_semantics=("arbitrary",),
            vmem_limit_bytes=100 * 1024 * 1024,
        ),
        interpret=interpret,
    )(x4)
    return out.reshape(_ROWS, _N)
